# Initial kernel scaffold; baseline (speedup 1.0000x reference)
import jax
import jax.numpy as jnp
from jax.experimental import pallas as pl


def kernel(x, edge_index, batch, W1_0, b1_0, W2_0, b2_0, eps_0, W1_1, b1_1, W2_1, b2_1, eps_1, W1_2, b1_2, W2_2, b2_2, eps_2, P1, pb1, P2, pb2):
    raise NotImplementedError("write your pallas kernel here")


# speedup vs baseline: 7.2840x; 7.2840x over previous
import functools

import jax
import jax.numpy as jnp
from jax import lax
from jax.experimental import pallas as pl
from jax.experimental.pallas import tpu as pltpu
from jax.experimental.pallas import tpu_sc as plsc

N = 10000
E = 320000
IN_DIM = 128
DIM = 32
NUM_GRAPHS = 128

NPAD = 10240
ROWS_PER_TILE = NPAD // 32
CHUNK = 128
NTILES = 32
CHUNKS_PER_TILE = 80
EPAD = NTILES * CHUNKS_PER_TILE * CHUNK


def _sc_agg_body(y_hbm, src_hbm, dst_hbm, zeros_hbm, out_hbm,
                 sidx, didx, rows, acc, sem):
    c = lax.axis_index("c")
    s = lax.axis_index("s")
    wid = c * 16 + s

    pltpu.sync_copy(zeros_hbm.at[pl.ds(s * ROWS_PER_TILE, ROWS_PER_TILE)],
                    acc.at[pl.ds(s * ROWS_PER_TILE, ROWS_PER_TILE)])

    pltpu.sync_copy(src_hbm.at[pl.ds(wid * CHUNKS_PER_TILE, CHUNKS_PER_TILE)], sidx)
    pltpu.sync_copy(dst_hbm.at[pl.ds(wid * CHUNKS_PER_TILE, CHUNKS_PER_TILE)], didx)

    plsc.subcore_barrier()

    def body(k, carry):
        pltpu.async_copy(y_hbm.at[sidx.at[k]], rows, sem).wait()
        pltpu.sync_copy(rows, acc.at[didx.at[k]], add=True)
        return carry

    lax.fori_loop(0, CHUNKS_PER_TILE, body, 0, unroll=False)

    plsc.subcore_barrier()

    pltpu.sync_copy(acc.at[pl.ds(s * ROWS_PER_TILE, ROWS_PER_TILE)],
                    out_hbm.at[c, pl.ds(s * ROWS_PER_TILE, ROWS_PER_TILE)])


_sc_agg = functools.partial(
    pl.kernel,
    out_type=jax.ShapeDtypeStruct((2, NPAD, DIM), jnp.float32),
    mesh=plsc.VectorSubcoreMesh(core_axis_name="c", subcore_axis_name="s"),
    scratch_types=[
        pltpu.VMEM((CHUNKS_PER_TILE, CHUNK), jnp.int32),
        pltpu.VMEM((CHUNKS_PER_TILE, CHUNK), jnp.int32),
        pltpu.VMEM((CHUNK, DIM), jnp.float32),
        pltpu.VMEM_SHARED((NPAD, DIM), jnp.float32),
        pltpu.SemaphoreType.DMA,
    ],
    compiler_params=pltpu.CompilerParams(use_tc_tiling_on_sc=False),
)(_sc_agg_body)


def _stage0_body(x_ref, w_ref, y_ref):
    y_ref[:N, :] = jnp.dot(x_ref[...], w_ref[...],
                           preferred_element_type=jnp.float32)
    y_ref[N:, :] = jnp.zeros((NPAD - N, DIM), jnp.float32)


def _onehot_pool(batch_ref, h):
    gid = lax.broadcasted_iota(jnp.int32, (NUM_GRAPHS, N), 0)
    onehot = (gid == batch_ref[...]).astype(jnp.float32)
    return jnp.dot(onehot, h, preferred_element_type=jnp.float32)


def _stage_mid_body(p_ref, y_ref, b1_ref, w2_ref, b2_ref, batch_ref, w1n_ref,
                    scale_ref, yn_ref, pool_ref):
    scale = scale_ref[0]
    y = y_ref[:N, :]
    z = p_ref[0, :N, :] + p_ref[1, :N, :] + scale * y + b1_ref[...]
    t = jnp.maximum(z, 0.0)
    h = jnp.maximum(jnp.dot(t, w2_ref[...], preferred_element_type=jnp.float32)
                    + b2_ref[...], 0.0)
    pool_ref[...] = _onehot_pool(batch_ref, h)
    yn_ref[:N, :] = jnp.dot(h, w1n_ref[...], preferred_element_type=jnp.float32)
    yn_ref[N:, :] = jnp.zeros((NPAD - N, DIM), jnp.float32)


def _stage_last_body(p_ref, y_ref, b1_ref, w2_ref, b2_ref, batch_ref,
                     pool0_ref, pool1_ref, p1a_ref, p1b_ref, p1c_ref,
                     pb1_ref, p2_ref, pb2_ref, scale_ref, out_ref):
    scale = scale_ref[0]
    y = y_ref[:N, :]
    z = p_ref[0, :N, :] + p_ref[1, :N, :] + scale * y + b1_ref[...]
    t = jnp.maximum(z, 0.0)
    h = jnp.maximum(jnp.dot(t, w2_ref[...], preferred_element_type=jnp.float32)
                    + b2_ref[...], 0.0)
    pool2 = _onehot_pool(batch_ref, h)
    g = (jnp.dot(pool0_ref[...], p1a_ref[...], preferred_element_type=jnp.float32)
         + jnp.dot(pool1_ref[...], p1b_ref[...], preferred_element_type=jnp.float32)
         + jnp.dot(pool2, p1c_ref[...], preferred_element_type=jnp.float32)
         + pb1_ref[...])
    g = jnp.maximum(g, 0.0)
    out_ref[...] = (jnp.dot(g, p2_ref[...], preferred_element_type=jnp.float32)
                    + pb2_ref[...])


_VMEM = pl.BlockSpec(memory_space=pltpu.VMEM)
_SMEM = pl.BlockSpec(memory_space=pltpu.SMEM)

_stage0 = pl.pallas_call(
    _stage0_body,
    out_shape=jax.ShapeDtypeStruct((NPAD, DIM), jnp.float32),
    in_specs=[_VMEM, _VMEM],
    out_specs=_VMEM,
)

_stage_mid = pl.pallas_call(
    _stage_mid_body,
    out_shape=(jax.ShapeDtypeStruct((NPAD, DIM), jnp.float32),
               jax.ShapeDtypeStruct((NUM_GRAPHS, DIM), jnp.float32)),
    in_specs=[_VMEM] * 7 + [_SMEM],
    out_specs=(_VMEM, _VMEM),
)

_stage_last = pl.pallas_call(
    _stage_last_body,
    out_shape=jax.ShapeDtypeStruct((NUM_GRAPHS, DIM), jnp.float32),
    in_specs=[_VMEM] * 14 + [_SMEM],
    out_specs=_VMEM,
)


def kernel(x, edge_index, batch, W1_0, b1_0, W2_0, b2_0, eps_0, W1_1, b1_1,
           W2_1, b2_1, eps_1, W1_2, b1_2, W2_2, b2_2, eps_2, P1, pb1, P2, pb2):
    edge_index = edge_index.astype(jnp.int32)
    src = jnp.concatenate(
        [edge_index[0], jnp.full((EPAD - E,), N, jnp.int32)])
    dst = jnp.concatenate(
        [edge_index[1], jnp.full((EPAD - E,), N, jnp.int32)])
    src2d = src.reshape(NTILES * CHUNKS_PER_TILE, CHUNK)
    dst2d = dst.reshape(NTILES * CHUNKS_PER_TILE, CHUNK)
    batch2d = batch.astype(jnp.int32).reshape(1, N)
    zeros = jnp.zeros((NPAD, DIM), jnp.float32)

    b1 = [b1_0.reshape(1, DIM), b1_1.reshape(1, DIM), b1_2.reshape(1, DIM)]
    b2 = [b2_0.reshape(1, DIM), b2_1.reshape(1, DIM), b2_2.reshape(1, DIM)]
    W2 = [W2_0, W2_1, W2_2]
    scales = [jnp.reshape(1.0 + eps_0, (1,)), jnp.reshape(1.0 + eps_1, (1,)),
              jnp.reshape(1.0 + eps_2, (1,))]

    y0 = _stage0(x, W1_0)
    p0 = _sc_agg(y0, src2d, dst2d, zeros)
    y1, pool0 = _stage_mid(p0, y0, b1[0], W2[0], b2[0], batch2d, W1_1,
                           scales[0])
    p1 = _sc_agg(y1, src2d, dst2d, zeros)
    y2, pool1 = _stage_mid(p1, y1, b1[1], W2[1], b2[1], batch2d, W1_2,
                           scales[1])
    p2 = _sc_agg(y2, src2d, dst2d, zeros)
    out = _stage_last(p2, y2, b1[2], W2[2], b2[2], batch2d, pool0, pool1,
                      P1[:DIM], P1[DIM:2 * DIM], P1[2 * DIM:],
                      pb1.reshape(1, DIM), P2, pb2.reshape(1, DIM), scales[2])
    return out


# speedup vs baseline: 9.3861x; 1.2886x over previous
import functools

import jax
import jax.numpy as jnp
from jax import lax
from jax.experimental import pallas as pl
from jax.experimental.pallas import tpu as pltpu
from jax.experimental.pallas import tpu_sc as plsc

N = 10000
E = 320000
IN_DIM = 128
DIM = 32
NUM_GRAPHS = 128

NPAD = 10240
ROWS_PER_TILE = NPAD // 32
CHUNK = 128
NTILES = 32
CHUNKS_PER_TILE = 80
EPAD = NTILES * CHUNKS_PER_TILE * CHUNK


NBUF = 4


def _sc_agg_body(y_hbm, src_hbm, dst_hbm, zeros_hbm, out_hbm,
                 sidx, didx, rows, acc, *gsems):
    c = lax.axis_index("c")
    s = lax.axis_index("s")
    wid = c * 16 + s

    pltpu.sync_copy(zeros_hbm.at[pl.ds(s * ROWS_PER_TILE, ROWS_PER_TILE)],
                    acc.at[pl.ds(s * ROWS_PER_TILE, ROWS_PER_TILE)])

    pltpu.sync_copy(src_hbm.at[pl.ds(wid * CHUNKS_PER_TILE, CHUNKS_PER_TILE)], sidx)
    pltpu.sync_copy(dst_hbm.at[pl.ds(wid * CHUNKS_PER_TILE, CHUNKS_PER_TILE)], didx)

    plsc.subcore_barrier()

    for b in range(NBUF):
        pltpu.async_copy(y_hbm.at[sidx.at[b]], rows.at[b], gsems[b])

    def body(k0, carry):
        for b in range(NBUF):
            k = k0 * NBUF + b
            pltpu.make_async_copy(y_hbm.at[sidx.at[k]], rows.at[b],
                                  gsems[b]).wait()
            pltpu.sync_copy(rows.at[b], acc.at[didx.at[k]], add=True)
            pltpu.async_copy(y_hbm.at[sidx.at[k + NBUF]], rows.at[b], gsems[b])
        return carry

    lax.fori_loop(0, CHUNKS_PER_TILE // NBUF - 1, body, 0, unroll=False)

    for b in range(NBUF):
        k = CHUNKS_PER_TILE - NBUF + b
        pltpu.make_async_copy(y_hbm.at[sidx.at[k]], rows.at[b],
                              gsems[b]).wait()
        pltpu.sync_copy(rows.at[b], acc.at[didx.at[k]], add=True)

    plsc.subcore_barrier()

    pltpu.sync_copy(acc.at[pl.ds(s * ROWS_PER_TILE, ROWS_PER_TILE)],
                    out_hbm.at[c, pl.ds(s * ROWS_PER_TILE, ROWS_PER_TILE)])


_sc_agg = functools.partial(
    pl.kernel,
    out_type=jax.ShapeDtypeStruct((2, NPAD, DIM), jnp.float32),
    mesh=plsc.VectorSubcoreMesh(core_axis_name="c", subcore_axis_name="s"),
    scratch_types=[
        pltpu.VMEM((CHUNKS_PER_TILE, CHUNK), jnp.int32),
        pltpu.VMEM((CHUNKS_PER_TILE, CHUNK), jnp.int32),
        pltpu.VMEM((NBUF, CHUNK, DIM), jnp.float32),
        pltpu.VMEM_SHARED((NPAD, DIM), jnp.float32),
    ] + [pltpu.SemaphoreType.DMA] * NBUF,
    compiler_params=pltpu.CompilerParams(use_tc_tiling_on_sc=False),
)(_sc_agg_body)


def _stage0_body(x_ref, w_ref, y_ref):
    y_ref[:N, :] = jnp.dot(x_ref[...], w_ref[...],
                           preferred_element_type=jnp.float32)
    y_ref[N:, :] = jnp.zeros((NPAD - N, DIM), jnp.float32)


def _onehot_pool(batch_ref, h):
    gid = lax.broadcasted_iota(jnp.int32, (NUM_GRAPHS, N), 0)
    onehot = (gid == batch_ref[...]).astype(jnp.float32)
    return jnp.dot(onehot, h, preferred_element_type=jnp.float32)


def _stage_mid_body(p_ref, y_ref, b1_ref, w2_ref, b2_ref, batch_ref, w1n_ref,
                    scale_ref, yn_ref, pool_ref):
    scale = scale_ref[0]
    y = y_ref[:N, :]
    z = p_ref[0, :N, :] + p_ref[1, :N, :] + scale * y + b1_ref[...]
    t = jnp.maximum(z, 0.0)
    h = jnp.maximum(jnp.dot(t, w2_ref[...], preferred_element_type=jnp.float32)
                    + b2_ref[...], 0.0)
    pool_ref[...] = _onehot_pool(batch_ref, h)
    yn_ref[:N, :] = jnp.dot(h, w1n_ref[...], preferred_element_type=jnp.float32)
    yn_ref[N:, :] = jnp.zeros((NPAD - N, DIM), jnp.float32)


def _stage_last_body(p_ref, y_ref, b1_ref, w2_ref, b2_ref, batch_ref,
                     pool0_ref, pool1_ref, p1a_ref, p1b_ref, p1c_ref,
                     pb1_ref, p2_ref, pb2_ref, scale_ref, out_ref):
    scale = scale_ref[0]
    y = y_ref[:N, :]
    z = p_ref[0, :N, :] + p_ref[1, :N, :] + scale * y + b1_ref[...]
    t = jnp.maximum(z, 0.0)
    h = jnp.maximum(jnp.dot(t, w2_ref[...], preferred_element_type=jnp.float32)
                    + b2_ref[...], 0.0)
    pool2 = _onehot_pool(batch_ref, h)
    g = (jnp.dot(pool0_ref[...], p1a_ref[...], preferred_element_type=jnp.float32)
         + jnp.dot(pool1_ref[...], p1b_ref[...], preferred_element_type=jnp.float32)
         + jnp.dot(pool2, p1c_ref[...], preferred_element_type=jnp.float32)
         + pb1_ref[...])
    g = jnp.maximum(g, 0.0)
    out_ref[...] = (jnp.dot(g, p2_ref[...], preferred_element_type=jnp.float32)
                    + pb2_ref[...])


_VMEM = pl.BlockSpec(memory_space=pltpu.VMEM)
_SMEM = pl.BlockSpec(memory_space=pltpu.SMEM)

_stage0 = pl.pallas_call(
    _stage0_body,
    out_shape=jax.ShapeDtypeStruct((NPAD, DIM), jnp.float32),
    in_specs=[_VMEM, _VMEM],
    out_specs=_VMEM,
)

_stage_mid = pl.pallas_call(
    _stage_mid_body,
    out_shape=(jax.ShapeDtypeStruct((NPAD, DIM), jnp.float32),
               jax.ShapeDtypeStruct((NUM_GRAPHS, DIM), jnp.float32)),
    in_specs=[_VMEM] * 7 + [_SMEM],
    out_specs=(_VMEM, _VMEM),
)

_stage_last = pl.pallas_call(
    _stage_last_body,
    out_shape=jax.ShapeDtypeStruct((NUM_GRAPHS, DIM), jnp.float32),
    in_specs=[_VMEM] * 14 + [_SMEM],
    out_specs=_VMEM,
)


def kernel(x, edge_index, batch, W1_0, b1_0, W2_0, b2_0, eps_0, W1_1, b1_1,
           W2_1, b2_1, eps_1, W1_2, b1_2, W2_2, b2_2, eps_2, P1, pb1, P2, pb2):
    edge_index = edge_index.astype(jnp.int32)
    src = jnp.concatenate(
        [edge_index[0], jnp.full((EPAD - E,), N, jnp.int32)])
    dst = jnp.concatenate(
        [edge_index[1], jnp.full((EPAD - E,), N, jnp.int32)])
    src2d = src.reshape(NTILES * CHUNKS_PER_TILE, CHUNK)
    dst2d = dst.reshape(NTILES * CHUNKS_PER_TILE, CHUNK)
    batch2d = batch.astype(jnp.int32).reshape(1, N)
    zeros = jnp.zeros((NPAD, DIM), jnp.float32)

    b1 = [b1_0.reshape(1, DIM), b1_1.reshape(1, DIM), b1_2.reshape(1, DIM)]
    b2 = [b2_0.reshape(1, DIM), b2_1.reshape(1, DIM), b2_2.reshape(1, DIM)]
    W2 = [W2_0, W2_1, W2_2]
    scales = [jnp.reshape(1.0 + eps_0, (1,)), jnp.reshape(1.0 + eps_1, (1,)),
              jnp.reshape(1.0 + eps_2, (1,))]

    y0 = _stage0(x, W1_0)
    p0 = _sc_agg(y0, src2d, dst2d, zeros)
    y1, pool0 = _stage_mid(p0, y0, b1[0], W2[0], b2[0], batch2d, W1_1,
                           scales[0])
    p1 = _sc_agg(y1, src2d, dst2d, zeros)
    y2, pool1 = _stage_mid(p1, y1, b1[1], W2[1], b2[1], batch2d, W1_2,
                           scales[1])
    p2 = _sc_agg(y2, src2d, dst2d, zeros)
    out = _stage_last(p2, y2, b1[2], W2[2], b2[2], batch2d, pool0, pool1,
                      P1[:DIM], P1[DIM:2 * DIM], P1[2 * DIM:],
                      pb1.reshape(1, DIM), P2, pb2.reshape(1, DIM), scales[2])
    return out


# speedup vs baseline: 22.4158x; 2.3882x over previous
import functools

import jax
import jax.numpy as jnp
from jax import lax
from jax.experimental import pallas as pl
from jax.experimental.pallas import tpu as pltpu
from jax.experimental.pallas import tpu_sc as plsc

N = 10000
E = 320000
IN_DIM = 128
DIM = 32
NUM_GRAPHS = 128

NPAD = 10240
ROWS_PER_TILE = NPAD // 32
CHUNK = 128
NTILES = 32
CHUNKS_PER_TILE = 80
EPAD = NTILES * CHUNKS_PER_TILE * CHUNK


NBUF = 4


def _sc_agg_body(y_hbm, src_hbm, dst_hbm, zeros_hbm, out_hbm,
                 sidx, didx, rows, acc, *gsems):
    c = lax.axis_index("c")
    s = lax.axis_index("s")
    wid = c * 16 + s

    pltpu.sync_copy(zeros_hbm.at[pl.ds(s * ROWS_PER_TILE, ROWS_PER_TILE)],
                    acc.at[pl.ds(s * ROWS_PER_TILE, ROWS_PER_TILE)])

    pltpu.sync_copy(src_hbm.at[pl.ds(wid * CHUNKS_PER_TILE, CHUNKS_PER_TILE)], sidx)
    pltpu.sync_copy(dst_hbm.at[pl.ds(wid * CHUNKS_PER_TILE, CHUNKS_PER_TILE)], didx)

    plsc.subcore_barrier()

    for b in range(NBUF):
        pltpu.async_copy(y_hbm.at[sidx.at[b]], rows.at[b], gsems[b])

    def body(k0, carry):
        for b in range(NBUF):
            k = k0 * NBUF + b
            pltpu.make_async_copy(y_hbm.at[sidx.at[k]], rows.at[b],
                                  gsems[b]).wait()
            pltpu.sync_copy(rows.at[b], acc.at[didx.at[k]], add=True)
            pltpu.async_copy(y_hbm.at[sidx.at[k + NBUF]], rows.at[b], gsems[b])
        return carry

    lax.fori_loop(0, CHUNKS_PER_TILE // NBUF - 1, body, 0, unroll=False)

    for b in range(NBUF):
        k = CHUNKS_PER_TILE - NBUF + b
        pltpu.make_async_copy(y_hbm.at[sidx.at[k]], rows.at[b],
                              gsems[b]).wait()
        pltpu.sync_copy(rows.at[b], acc.at[didx.at[k]], add=True)

    plsc.subcore_barrier()

    pltpu.sync_copy(acc.at[pl.ds(s * ROWS_PER_TILE, ROWS_PER_TILE)],
                    out_hbm.at[c, pl.ds(s * ROWS_PER_TILE, ROWS_PER_TILE)])


_sc_agg = functools.partial(
    pl.kernel,
    out_type=jax.ShapeDtypeStruct((2, NPAD, DIM), jnp.float32),
    mesh=plsc.VectorSubcoreMesh(core_axis_name="c", subcore_axis_name="s"),
    scratch_types=[
        pltpu.VMEM((CHUNKS_PER_TILE, CHUNK), jnp.int32),
        pltpu.VMEM((CHUNKS_PER_TILE, CHUNK), jnp.int32),
        pltpu.VMEM((NBUF, CHUNK, DIM), jnp.float32),
        pltpu.VMEM_SHARED((NPAD, DIM), jnp.float32),
    ] + [pltpu.SemaphoreType.DMA] * NBUF,
    compiler_params=pltpu.CompilerParams(use_tc_tiling_on_sc=False),
)(_sc_agg_body)


def _stage0_body(x_ref, w_ref, y_ref):
    y_ref[:N, :] = jnp.dot(x_ref[...], w_ref[...],
                           preferred_element_type=jnp.float32)
    y_ref[N:, :] = jnp.zeros((NPAD - N, DIM), jnp.float32)


def _onehot_pool(batch_ref, h):
    gid = lax.broadcasted_iota(jnp.int32, (NUM_GRAPHS, N), 0)
    onehot = (gid == batch_ref[...]).astype(jnp.float32)
    return jnp.dot(onehot, h, preferred_element_type=jnp.float32)


def _stage_mid_body(p_ref, y_ref, b1_ref, w2_ref, b2_ref, batch_ref, w1n_ref,
                    scale_ref, yn_ref, pool_ref):
    scale = scale_ref[0]
    y = y_ref[:N, :]
    z = p_ref[0, :N, :] + p_ref[1, :N, :] + scale * y + b1_ref[...]
    t = jnp.maximum(z, 0.0)
    h = jnp.maximum(jnp.dot(t, w2_ref[...], preferred_element_type=jnp.float32)
                    + b2_ref[...], 0.0)
    pool_ref[...] = _onehot_pool(batch_ref, h)
    yn_ref[:N, :] = jnp.dot(h, w1n_ref[...], preferred_element_type=jnp.float32)
    yn_ref[N:, :] = jnp.zeros((NPAD - N, DIM), jnp.float32)


def _stage_last_body(p_ref, y_ref, b1_ref, w2_ref, b2_ref, batch_ref,
                     pool0_ref, pool1_ref, p1a_ref, p1b_ref, p1c_ref,
                     pb1_ref, p2_ref, pb2_ref, scale_ref, out_ref):
    scale = scale_ref[0]
    y = y_ref[:N, :]
    z = p_ref[0, :N, :] + p_ref[1, :N, :] + scale * y + b1_ref[...]
    t = jnp.maximum(z, 0.0)
    h = jnp.maximum(jnp.dot(t, w2_ref[...], preferred_element_type=jnp.float32)
                    + b2_ref[...], 0.0)
    pool2 = _onehot_pool(batch_ref, h)
    g = (jnp.dot(pool0_ref[...], p1a_ref[...], preferred_element_type=jnp.float32)
         + jnp.dot(pool1_ref[...], p1b_ref[...], preferred_element_type=jnp.float32)
         + jnp.dot(pool2, p1c_ref[...], preferred_element_type=jnp.float32)
         + pb1_ref[...])
    g = jnp.maximum(g, 0.0)
    out_ref[...] = (jnp.dot(g, p2_ref[...], preferred_element_type=jnp.float32)
                    + pb2_ref[...])


_VMEM = pl.BlockSpec(memory_space=pltpu.VMEM)
_SMEM = pl.BlockSpec(memory_space=pltpu.SMEM)

_stage0 = pl.pallas_call(
    _stage0_body,
    out_shape=jax.ShapeDtypeStruct((NPAD, DIM), jnp.float32),
    in_specs=[_VMEM, _VMEM],
    out_specs=_VMEM,
)

_stage_mid = pl.pallas_call(
    _stage_mid_body,
    out_shape=(jax.ShapeDtypeStruct((NPAD, DIM), jnp.float32),
               jax.ShapeDtypeStruct((NUM_GRAPHS, DIM), jnp.float32)),
    in_specs=[_VMEM] * 7 + [_SMEM],
    out_specs=(_VMEM, _VMEM),
)

_stage_last = pl.pallas_call(
    _stage_last_body,
    out_shape=jax.ShapeDtypeStruct((NUM_GRAPHS, DIM), jnp.float32),
    in_specs=[_VMEM] * 14 + [_SMEM],
    out_specs=_VMEM,
)


def kernel(x, edge_index, batch, W1_0, b1_0, W2_0, b2_0, eps_0, W1_1, b1_1,
           W2_1, b2_1, eps_1, W1_2, b1_2, W2_2, b2_2, eps_2, P1, pb1, P2, pb2):
    edge_index = edge_index.astype(jnp.int32)
    pad_rows = N + jnp.arange(EPAD - E, dtype=jnp.int32) % (NPAD - N)
    src = jnp.concatenate([edge_index[0], pad_rows])
    dst = jnp.concatenate([edge_index[1], pad_rows])
    src2d = src.reshape(NTILES * CHUNKS_PER_TILE, CHUNK)
    dst2d = dst.reshape(NTILES * CHUNKS_PER_TILE, CHUNK)
    batch2d = batch.astype(jnp.int32).reshape(1, N)
    zeros = jnp.zeros((NPAD, DIM), jnp.float32)

    b1 = [b1_0.reshape(1, DIM), b1_1.reshape(1, DIM), b1_2.reshape(1, DIM)]
    b2 = [b2_0.reshape(1, DIM), b2_1.reshape(1, DIM), b2_2.reshape(1, DIM)]
    W2 = [W2_0, W2_1, W2_2]
    scales = [jnp.reshape(1.0 + eps_0, (1,)), jnp.reshape(1.0 + eps_1, (1,)),
              jnp.reshape(1.0 + eps_2, (1,))]

    y0 = _stage0(x, W1_0)
    p0 = _sc_agg(y0, src2d, dst2d, zeros)
    y1, pool0 = _stage_mid(p0, y0, b1[0], W2[0], b2[0], batch2d, W1_1,
                           scales[0])
    p1 = _sc_agg(y1, src2d, dst2d, zeros)
    y2, pool1 = _stage_mid(p1, y1, b1[1], W2[1], b2[1], batch2d, W1_2,
                           scales[1])
    p2 = _sc_agg(y2, src2d, dst2d, zeros)
    out = _stage_last(p2, y2, b1[2], W2[2], b2[2], batch2d, pool0, pool1,
                      P1[:DIM], P1[DIM:2 * DIM], P1[2 * DIM:],
                      pb1.reshape(1, DIM), P2, pb2.reshape(1, DIM), scales[2])
    return out
